# baseline (speedup 1.0000x reference)
import functools

import jax
import jax.numpy as jnp
from jax import lax
from jax.experimental import pallas as pl
from jax.experimental.pallas import tpu as pltpu
from jax.experimental.pallas import tpu_sc as plsc

T, H, F, E = 2048, 768, 1024, 8
BM = 256
TP = T * 2 + E * BM
NT = TP // BM
RW = 32
NR = TP // RW
NSC = 16
TOK_PER_TILE = T // NSC


def _router_body(x_ref, wgt_ref, logits_ref, post_ref, wt_ref, te_ref):
    x = x_ref[...]
    logits = jnp.dot(x, wgt_ref[...], preferred_element_type=jnp.float32)
    logits_ref[...] = logits
    m = jnp.max(logits, axis=1, keepdims=True)
    ex = jnp.exp(logits - m)
    p = ex / jnp.sum(ex, axis=1, keepdims=True)
    idx = lax.broadcasted_iota(jnp.int32, p.shape, 1)
    m1 = jnp.max(p, axis=1, keepdims=True)
    a1 = jnp.min(jnp.where(p == m1, idx, E), axis=1, keepdims=True)
    mask1 = idx == a1
    p2 = jnp.where(mask1, -1.0, p)
    m2 = jnp.max(p2, axis=1, keepdims=True)
    a2 = jnp.min(jnp.where(p2 == m2, idx, E), axis=1, keepdims=True)
    mask2 = idx == a2
    s = m1 + m2
    occ = jnp.where(mask1 | mask2, 1.0, 0.0)
    cum = occ
    k = 1
    while k < T:
        shifted = jnp.concatenate([jnp.zeros((k, E), jnp.float32), cum[:-k]], 0)
        cum = cum + shifted
        k *= 2
    total = cum[T - 1:T, :]
    cume = cum - occ
    pt = jnp.floor((total + (BM - 1)) / BM) * BM
    ei = lax.broadcasted_iota(jnp.int32, (E, E), 0)
    ej = lax.broadcasted_iota(jnp.int32, (E, E), 1)
    strict_lt = jnp.where(ei < ej, 1.0, 0.0)
    po = jnp.dot(pt, strict_lt, preferred_element_type=jnp.float32)
    pos = po + cume
    pos0 = jnp.sum(jnp.where(mask1, pos, 0.0), axis=1, keepdims=True)
    pos1 = jnp.sum(jnp.where(mask2, pos, 0.0), axis=1, keepdims=True)
    post_ref[0:1, :] = pos0.reshape(1, T).astype(jnp.int32)
    post_ref[1:2, :] = pos1.reshape(1, T).astype(jnp.int32)
    wt_ref[0:1, :] = (m1 / s).reshape(1, T)
    wt_ref[1:2, :] = (m2 / s).reshape(1, T)
    mb = (lax.broadcasted_iota(jnp.int32, (NT, E), 0) * BM).astype(jnp.float32)
    cnt = jnp.sum(jnp.where(jnp.broadcast_to(po, (NT, E)) <= mb, 1, 0),
                  axis=1, keepdims=True)
    te_ref[...] = (cnt - 1).reshape(1, NT)


def _iota16(off):
    return lax.broadcasted_iota(jnp.int32, (16,), 0) + off


def _dispatch_gather_body(post, wt, x, xs, sortw,
                          zbuf_i, zbuf_f, posbuf0, posbuf1, wbuf0, wbuf1,
                          tokbuf, idxbuf, rows_a, rows_b,
                          shared_tok, shared_w,
                          sem, scsem, gsem, wsem):
    c = lax.axis_index("c")
    sid = lax.axis_index("s")

    def zbody(i, carry):
        zbuf_i[pl.ds(i * 16, 16)] = jnp.zeros((16,), jnp.int32)
        zbuf_f[pl.ds(i * 16, 16)] = jnp.zeros((16,), jnp.float32)
        return carry

    lax.fori_loop(0, (TP // NSC) // 16, zbody, 0)
    zn = TP // NSC
    pltpu.sync_copy(zbuf_i, shared_tok.at[pl.ds(zn * sid, zn)])
    pltpu.sync_copy(zbuf_f, shared_w.at[pl.ds(zn * sid, zn)])

    base = sid * TOK_PER_TILE
    pltpu.sync_copy(post.at[0, pl.ds(base, TOK_PER_TILE)], posbuf0)
    pltpu.sync_copy(post.at[1, pl.ds(base, TOK_PER_TILE)], posbuf1)
    pltpu.sync_copy(wt.at[0, pl.ds(base, TOK_PER_TILE)], wbuf0)
    pltpu.sync_copy(wt.at[1, pl.ds(base, TOK_PER_TILE)], wbuf1)
    for k in range(TOK_PER_TILE // 16):
        tokbuf[pl.ds(16 * k, 16)] = _iota16(base + 16 * k)

    plsc.subcore_barrier()
    copies = [
        pltpu.async_copy(tokbuf, shared_tok.at[posbuf0], scsem, add=True),
        pltpu.async_copy(tokbuf, shared_tok.at[posbuf1], scsem, add=True),
        pltpu.async_copy(wbuf0, shared_w.at[posbuf0], scsem, add=True),
        pltpu.async_copy(wbuf1, shared_w.at[posbuf1], scsem, add=True),
    ]
    for cp in copies:
        cp.wait()
    plsc.subcore_barrier()

    n_share = TP // 256

    @pl.when(sid < n_share // 2)
    def _writeback_gather():
        share = 2 * sid + c
        e0 = 256 * share
        pltpu.sync_copy(shared_w.at[pl.ds(e0, 256)],
                        sortw.at[pl.ds(e0, 256)])
        pltpu.sync_copy(shared_tok.at[pl.ds(e0, 256)], idxbuf)
        bufs = [rows_a, rows_b]
        gcp = [None] * 4
        wcp = [None] * 4
        gcp[0] = pltpu.async_copy(x.at[idxbuf.at[pl.ds(0, 64)]], rows_a, gsem)
        gcp[1] = pltpu.async_copy(x.at[idxbuf.at[pl.ds(64, 64)]], rows_b, gsem)
        for ch in range(4):
            gcp[ch].wait()
            wcp[ch] = pltpu.async_copy(
                bufs[ch % 2], xs.at[pl.ds(e0 + 64 * ch, 64), :], wsem)
            if ch + 2 < 4:
                wcp[ch].wait()
                gcp[ch + 2] = pltpu.async_copy(
                    x.at[idxbuf.at[pl.ds(64 * (ch + 2), 64)]],
                    bufs[ch % 2], gsem)
        wcp[2].wait()
        wcp[3].wait()


def _moe_body(te_ref, xs_ref, w1_ref, w3_ref, w2_ref, sw_ref, out_ref):
    x = xs_ref[...].astype(jnp.bfloat16)
    w1 = w1_ref[0].astype(jnp.bfloat16)
    w3 = w3_ref[0].astype(jnp.bfloat16)
    w2 = w2_ref[0].astype(jnp.bfloat16)
    gate = jnp.dot(x, w1.T, preferred_element_type=jnp.float32)
    up = jnp.dot(x, w3.T, preferred_element_type=jnp.float32)
    act = (gate * jax.nn.sigmoid(gate) * up).astype(jnp.bfloat16)
    y = jnp.dot(act, w2.T, preferred_element_type=jnp.float32)
    out_ref[...] = y * sw_ref[0]


def _combine_body(post, ysw, out, i0, i1, buf_a, buf_b, sem, gsem, wsem):
    c = lax.axis_index("c")
    sid = lax.axis_index("s")

    @pl.when(sid < 8)
    def _do():
        share = 2 * sid + c
        base = 128 * share
        stage = [
            pltpu.async_copy(post.at[0, pl.ds(base, 128)], i0, sem),
            pltpu.async_copy(post.at[1, pl.ds(base, 128)], i1, sem),
        ]
        for cp in stage:
            cp.wait()
        for u in range(2):
            ga = pltpu.async_copy(ysw.at[i0.at[pl.ds(64 * u, 64)]], buf_a, gsem)
            gb = pltpu.async_copy(ysw.at[i1.at[pl.ds(64 * u, 64)]], buf_b, gsem)
            ga.wait()
            gb.wait()

            def body(r, carry):
                for cc in range(H // 16):
                    buf_a[r, pl.ds(16 * cc, 16)] = (
                        buf_a[r, pl.ds(16 * cc, 16)]
                        + buf_b[r, pl.ds(16 * cc, 16)])
                return carry

            lax.fori_loop(0, 64, body, 0)
            pltpu.sync_copy(buf_a, out.at[pl.ds(base + 64 * u, 64), :])


def kernel(hidden_states, Wg, W1, W3, W2):
    b, s, h = hidden_states.shape
    x = hidden_states.reshape(T, H)

    router = pl.pallas_call(
        _router_body,
        out_shape=[
            jax.ShapeDtypeStruct((T, E), jnp.float32),
            jax.ShapeDtypeStruct((2, T), jnp.int32),
            jax.ShapeDtypeStruct((2, T), jnp.float32),
            jax.ShapeDtypeStruct((1, NT), jnp.int32),
        ],
    )
    router_logits, post, wt, te2d = router(x, Wg.T)

    mesh = plsc.VectorSubcoreMesh(core_axis_name="c", subcore_axis_name="s")
    dispatch = functools.partial(
        pl.kernel,
        out_type=[
            jax.ShapeDtypeStruct((TP, H), jnp.float32),
            jax.ShapeDtypeStruct((TP,), jnp.float32),
        ],
        mesh=mesh,
        scratch_types=[
            pltpu.VMEM((TP // NSC,), jnp.int32),
            pltpu.VMEM((TP // NSC,), jnp.float32),
            pltpu.VMEM((TOK_PER_TILE,), jnp.int32),
            pltpu.VMEM((TOK_PER_TILE,), jnp.int32),
            pltpu.VMEM((TOK_PER_TILE,), jnp.float32),
            pltpu.VMEM((TOK_PER_TILE,), jnp.float32),
            pltpu.VMEM((TOK_PER_TILE,), jnp.int32),
            pltpu.VMEM((256,), jnp.int32),
            pltpu.VMEM((64, H), jnp.float32),
            pltpu.VMEM((64, H), jnp.float32),
            pltpu.VMEM_SHARED((TP,), jnp.int32),
            pltpu.VMEM_SHARED((TP,), jnp.float32),
            pltpu.SemaphoreType.DMA,
            pltpu.SemaphoreType.DMA,
            pltpu.SemaphoreType.DMA,
            pltpu.SemaphoreType.DMA,
        ],
    )(_dispatch_gather_body)
    xs, sortw = dispatch(post, wt, x)

    moe = pl.pallas_call(
        _moe_body,
        grid_spec=pltpu.PrefetchScalarGridSpec(
            num_scalar_prefetch=1,
            grid=(NT,),
            in_specs=[
                pl.BlockSpec((BM, H), lambda m, te: (m, 0)),
                pl.BlockSpec((1, F, H), lambda m, te: (te[m], 0, 0)),
                pl.BlockSpec((1, F, H), lambda m, te: (te[m], 0, 0)),
                pl.BlockSpec((1, H, F), lambda m, te: (te[m], 0, 0)),
                pl.BlockSpec((1, BM, 1), lambda m, te: (m, 0, 0)),
            ],
            out_specs=pl.BlockSpec((BM, H), lambda m, te: (m, 0)),
        ),
        out_shape=jax.ShapeDtypeStruct((TP, H), jnp.float32),
        compiler_params=pltpu.CompilerParams(
            dimension_semantics=("arbitrary",),
        ),
    )
    ysw = moe(te2d.reshape(NT), xs, W1, W3, W2, sortw.reshape(NT, BM, 1))

    combine = functools.partial(
        pl.kernel,
        out_type=jax.ShapeDtypeStruct((T, H), jnp.float32),
        mesh=mesh,
        scratch_types=[
            pltpu.VMEM((128,), jnp.int32),
            pltpu.VMEM((128,), jnp.int32),
            pltpu.VMEM((64, H), jnp.float32),
            pltpu.VMEM((64, H), jnp.float32),
            pltpu.SemaphoreType.DMA,
            pltpu.SemaphoreType.DMA,
            pltpu.SemaphoreType.DMA,
        ],
    )(_combine_body)
    out = combine(post, ysw)
    return out.reshape(b, s, h), router_logits


# speedup vs baseline: 2.1058x; 2.1058x over previous
import functools

import jax
import jax.numpy as jnp
from jax.experimental import pallas as pl
from jax.experimental.pallas import tpu as pltpu


def _router_body(x_ref, wgt_ref, logits_ref, dw_ref):
    x = x_ref[...]
    wgt = wgt_ref[...]
    logits = jnp.dot(x, wgt, preferred_element_type=jnp.float32)
    logits_ref[...] = logits
    e = logits.shape[1]
    m = jnp.max(logits, axis=1, keepdims=True)
    ex = jnp.exp(logits - m)
    p = ex / jnp.sum(ex, axis=1, keepdims=True)
    idx = jax.lax.broadcasted_iota(jnp.int32, p.shape, 1)
    m1 = jnp.max(p, axis=1, keepdims=True)
    a1 = jnp.min(jnp.where(p == m1, idx, e), axis=1, keepdims=True)
    mask1 = idx == a1
    p2 = jnp.where(mask1, -1.0, p)
    m2 = jnp.max(p2, axis=1, keepdims=True)
    a2 = jnp.min(jnp.where(p2 == m2, idx, e), axis=1, keepdims=True)
    mask2 = idx == a2
    dw_ref[...] = jnp.where(mask1 | mask2, p, 0.0) / (m1 + m2)


def _moe_body(x_ref, w1_ref, w3_ref, w2_ref, dw_ref, out_ref, acc_ref):
    e = pl.program_id(1)
    n_e = pl.num_programs(1)
    x = x_ref[...].astype(jnp.bfloat16)
    w1 = w1_ref[0].astype(jnp.bfloat16)
    w3 = w3_ref[0].astype(jnp.bfloat16)
    w2 = w2_ref[0].astype(jnp.bfloat16)
    gate = jnp.dot(x, w1.T, preferred_element_type=jnp.float32)
    up = jnp.dot(x, w3.T, preferred_element_type=jnp.float32)
    act = (gate * jax.nn.sigmoid(gate) * up).astype(jnp.bfloat16)
    y = jnp.dot(act, w2.T, preferred_element_type=jnp.float32)
    dw = dw_ref[...]
    eidx = jax.lax.broadcasted_iota(jnp.int32, dw.shape, 1)
    w_e = jnp.sum(jnp.where(eidx == e, dw, 0.0), axis=1, keepdims=True)
    contrib = y * w_e

    @pl.when(e == 0)
    def _init():
        acc_ref[...] = contrib

    @pl.when(e != 0)
    def _acc():
        acc_ref[...] = acc_ref[...] + contrib

    @pl.when(e == n_e - 1)
    def _out():
        out_ref[...] = acc_ref[...]


def kernel(hidden_states, Wg, W1, W3, W2):
    b, s, h = hidden_states.shape
    t = b * s
    e, f, _ = W1.shape
    x = hidden_states.reshape(t, h)

    bt_r = 256
    router = pl.pallas_call(
        _router_body,
        grid=(t // bt_r,),
        in_specs=[
            pl.BlockSpec((bt_r, h), lambda i: (i, 0)),
            pl.BlockSpec((h, e), lambda i: (0, 0)),
        ],
        out_specs=[
            pl.BlockSpec((bt_r, e), lambda i: (i, 0)),
            pl.BlockSpec((bt_r, e), lambda i: (i, 0)),
        ],
        out_shape=[
            jax.ShapeDtypeStruct((t, e), jnp.float32),
            jax.ShapeDtypeStruct((t, e), jnp.float32),
        ],
    )
    router_logits, dense_w = router(x, Wg.T)

    bm = 1024
    moe = pl.pallas_call(
        _moe_body,
        grid=(t // bm, e),
        in_specs=[
            pl.BlockSpec((bm, h), lambda ti, ei: (ti, 0)),
            pl.BlockSpec((1, f, h), lambda ti, ei: (ei, 0, 0)),
            pl.BlockSpec((1, f, h), lambda ti, ei: (ei, 0, 0)),
            pl.BlockSpec((1, h, f), lambda ti, ei: (ei, 0, 0)),
            pl.BlockSpec((bm, e), lambda ti, ei: (ti, 0)),
        ],
        out_specs=pl.BlockSpec((bm, h), lambda ti, ei: (ti, 0)),
        out_shape=jax.ShapeDtypeStruct((t, h), jnp.float32),
        scratch_shapes=[pltpu.VMEM((bm, h), jnp.float32)],
        compiler_params=pltpu.CompilerParams(
            dimension_semantics=("arbitrary", "arbitrary"),
        ),
    )
    out = moe(x, W1, W3, W2, dense_w)
    return out.reshape(b, s, h), router_logits


# speedup vs baseline: 2.1312x; 1.0121x over previous
import functools

import jax
import jax.numpy as jnp
from jax.experimental import pallas as pl
from jax.experimental.pallas import tpu as pltpu


def _router_body(x_ref, wgt_ref, logits_ref, dw_ref):
    x = x_ref[...]
    wgt = wgt_ref[...]
    logits = jnp.dot(x, wgt, preferred_element_type=jnp.float32)
    logits_ref[...] = logits
    e = logits.shape[1]
    m = jnp.max(logits, axis=1, keepdims=True)
    ex = jnp.exp(logits - m)
    p = ex / jnp.sum(ex, axis=1, keepdims=True)
    idx = jax.lax.broadcasted_iota(jnp.int32, p.shape, 1)
    m1 = jnp.max(p, axis=1, keepdims=True)
    a1 = jnp.min(jnp.where(p == m1, idx, e), axis=1, keepdims=True)
    mask1 = idx == a1
    p2 = jnp.where(mask1, -1.0, p)
    m2 = jnp.max(p2, axis=1, keepdims=True)
    a2 = jnp.min(jnp.where(p2 == m2, idx, e), axis=1, keepdims=True)
    mask2 = idx == a2
    dw_ref[...] = jnp.where(mask1 | mask2, p, 0.0) / (m1 + m2)


def _moe_body(x_ref, w1_ref, w3_ref, w2_ref, dw_ref, out_ref, acc_ref):
    e = pl.program_id(1)
    n_e = pl.num_programs(1)
    x = x_ref[...]
    w1 = w1_ref[0]
    w3 = w3_ref[0]
    w2 = w2_ref[0]
    gate = jnp.dot(x, w1.T, preferred_element_type=jnp.float32)
    up = jnp.dot(x, w3.T, preferred_element_type=jnp.float32)
    act = gate * jax.nn.sigmoid(gate) * up
    y = jnp.dot(act, w2.T, preferred_element_type=jnp.float32)
    dw = dw_ref[...]
    eidx = jax.lax.broadcasted_iota(jnp.int32, dw.shape, 1)
    w_e = jnp.sum(jnp.where(eidx == e, dw, 0.0), axis=1, keepdims=True)
    contrib = y * w_e

    @pl.when(e == 0)
    def _init():
        acc_ref[...] = contrib

    @pl.when(e != 0)
    def _acc():
        acc_ref[...] = acc_ref[...] + contrib

    @pl.when(e == n_e - 1)
    def _out():
        out_ref[...] = acc_ref[...]


def kernel(hidden_states, Wg, W1, W3, W2):
    b, s, h = hidden_states.shape
    t = b * s
    e, f, _ = W1.shape
    x = hidden_states.reshape(t, h)

    bt_r = 256
    router = pl.pallas_call(
        _router_body,
        grid=(t // bt_r,),
        in_specs=[
            pl.BlockSpec((bt_r, h), lambda i: (i, 0)),
            pl.BlockSpec((h, e), lambda i: (0, 0)),
        ],
        out_specs=[
            pl.BlockSpec((bt_r, e), lambda i: (i, 0)),
            pl.BlockSpec((bt_r, e), lambda i: (i, 0)),
        ],
        out_shape=[
            jax.ShapeDtypeStruct((t, e), jnp.float32),
            jax.ShapeDtypeStruct((t, e), jnp.float32),
        ],
    )
    router_logits, dense_w = router(x, Wg.T)

    bm = 1024
    moe = pl.pallas_call(
        _moe_body,
        grid=(t // bm, e),
        in_specs=[
            pl.BlockSpec((bm, h), lambda ti, ei: (ti, 0)),
            pl.BlockSpec((1, f, h), lambda ti, ei: (ei, 0, 0)),
            pl.BlockSpec((1, f, h), lambda ti, ei: (ei, 0, 0)),
            pl.BlockSpec((1, h, f), lambda ti, ei: (ei, 0, 0)),
            pl.BlockSpec((bm, e), lambda ti, ei: (ti, 0)),
        ],
        out_specs=pl.BlockSpec((bm, h), lambda ti, ei: (ti, 0)),
        out_shape=jax.ShapeDtypeStruct((t, h), jnp.float32),
        scratch_shapes=[pltpu.VMEM((bm, h), jnp.float32)],
        compiler_params=pltpu.CompilerParams(
            dimension_semantics=("arbitrary", "arbitrary"),
        ),
    )
    out = moe(x, W1, W3, W2, dense_w)
    return out.reshape(b, s, h), router_logits
